# baseline (speedup 1.0000x reference)
import functools

import jax
import jax.numpy as jnp
from jax import lax
from jax.experimental import pallas as pl
from jax.experimental.pallas import tpu as pltpu
from jax.experimental.pallas import tpu_sc as plsc

B, S, H, D = 1, 2048, 12, 64
DMODEL = H * D
DFF = 2048
E = 8
T = B * S
TOPK = 2

BT = 512
NTILES = (T * TOPK + E * (BT - 1) + BT - 1) // BT
LPAD = NTILES * BT
BF = 1024
NJ = DFF // BF

NC, NS = 2, 16
NW = NC * NS


def _router_kernel(x_ref, gw_ref, w_ref, idx_ref):
    x = x_ref[...]
    logits = jnp.dot(x, gw_ref[...], preferred_element_type=jnp.float32)
    am1 = jnp.argmax(logits, axis=1)[:, None]
    eids = jax.lax.broadcasted_iota(jnp.int32, logits.shape, 1)
    m1 = jnp.max(logits, axis=1, keepdims=True)
    masked = jnp.where(eids == am1, -jnp.inf, logits)
    am2 = jnp.argmax(masked, axis=1)[:, None]
    m2 = jnp.max(masked, axis=1, keepdims=True)
    w1v = 1.0 / (1.0 + jnp.exp(m2 - m1))
    w_ref[...] = jnp.concatenate([w1v, 1.0 - w1v], axis=1)
    idx_ref[...] = jnp.concatenate([am1, am2], axis=1).astype(jnp.int32)


def _router(x, gate_W):
    return pl.pallas_call(
        _router_kernel,
        out_shape=(jax.ShapeDtypeStruct((T, TOPK), jnp.float32),
                   jax.ShapeDtypeStruct((T, TOPK), jnp.int32)),
    )(x, gate_W)


def _dispatch(x, pos3):
    tok_w = T // NW

    mesh = plsc.VectorSubcoreMesh(core_axis_name="c", subcore_axis_name="s")

    @functools.partial(
        pl.kernel, mesh=mesh,
        out_type=jax.ShapeDtypeStruct((LPAD, DMODEL), jnp.float32),
        scratch_types=[
            pltpu.VMEM((TOPK, tok_w), jnp.int32),
            pltpu.VMEM((tok_w, DMODEL), jnp.float32),
            pltpu.SemaphoreType.DMA,
        ],
    )
    def k(x_hbm, pos3_hbm, xg_hbm, idx_v, rows_v, sem):
        wid = lax.axis_index("s") * NC + lax.axis_index("c")
        base = wid * tok_w
        pltpu.sync_copy(pos3_hbm.at[wid], idx_v)
        pltpu.sync_copy(x_hbm.at[pl.ds(base, tok_w)], rows_v)
        c0 = pltpu.async_copy(rows_v, xg_hbm.at[idx_v.at[0]], sem)
        c1 = pltpu.async_copy(rows_v, xg_hbm.at[idx_v.at[1]], sem)
        c0.wait()
        c1.wait()

    return k(x, pos3)


HALF = DMODEL // 2


def _ffn_kernel(te_ref, xg_ref, w1_ref, w3_ref, w2_ref, tokw_ref,
                out_ref, g_acc_ref, u_acc_ref, acc_ref):
    j = pl.program_id(0)
    tl = pl.program_id(1)
    kk = pl.program_id(2)
    keff = (tl + kk) % 2
    nvalid = te_ref[NTILES]

    @pl.when((j == 0) & (tl == 0) & (kk == 0))
    def _():
        out_ref[...] = jnp.zeros_like(out_ref)

    @pl.when(tl < nvalid)
    def _():
        xr = xg_ref[:, pl.ds(keff * HALF, HALF)]
        xb = jnp.where(jnp.abs(xr) < 1e30, xr, 0.0).astype(jnp.bfloat16)
        gp = jnp.dot(xb, w1_ref[0].astype(jnp.bfloat16),
                     preferred_element_type=jnp.float32)
        up = jnp.dot(xb, w3_ref[0].astype(jnp.bfloat16),
                     preferred_element_type=jnp.float32)

        @pl.when(kk == 0)
        def _():
            g_acc_ref[...] = gp
            u_acc_ref[...] = up

        @pl.when(kk == 1)
        def _():
            g = g_acc_ref[...] + gp
            u = u_acc_ref[...] + up
            g = g * jax.nn.sigmoid(g)
            h = (g * u).astype(jnp.bfloat16)
            part = jnp.dot(h, w2_ref[0].astype(jnp.bfloat16),
                           preferred_element_type=jnp.float32)

            sl = pl.ds(tl * BT, BT)
            if NJ > 1:
                @pl.when(j == 0)
                def _():
                    acc_ref[sl, :] = part.astype(jnp.bfloat16)

                @pl.when((j > 0) & (j < NJ - 1))
                def _():
                    acc_ref[sl, :] += part.astype(jnp.bfloat16)

            @pl.when(j == NJ - 1)
            def _():
                if NJ == 1:
                    full = part
                else:
                    full = acc_ref[sl, :].astype(jnp.float32) + part
                y = full.astype(jnp.bfloat16)
                v = tokw_ref[0]
                tok = v & 0xFFFF
                wv = (v >> 16).astype(jnp.float32) * (1.0 / 16384.0)
                ti = jax.lax.broadcasted_iota(jnp.int32, (T, BT), 0)
                pt = jnp.where(ti == tok, wv, 0.0).astype(jnp.bfloat16)
                out_ref[...] += jnp.dot(pt, y,
                                        preferred_element_type=jnp.float32)


def _ffn(scalars, xg, W1, W3, W2, tokw3):
    grid_spec = pltpu.PrefetchScalarGridSpec(
        num_scalar_prefetch=1,
        grid=(NJ, NTILES, 2),
        in_specs=[
            pl.BlockSpec((BT, DMODEL), lambda j, tl, kk, te: (tl, 0)),
            pl.BlockSpec((1, HALF, BF),
                         lambda j, tl, kk, te: (te[tl], (tl + kk) % 2, j)),
            pl.BlockSpec((1, HALF, BF),
                         lambda j, tl, kk, te: (te[tl], (tl + kk) % 2, j)),
            pl.BlockSpec((1, BF, DMODEL), lambda j, tl, kk, te: (te[tl], j, 0)),
            pl.BlockSpec((1, 1, BT), lambda j, tl, kk, te: (tl, 0, 0)),
        ],
        out_specs=pl.BlockSpec((T, DMODEL), lambda j, tl, kk, te: (0, 0)),
        scratch_shapes=[
            pltpu.VMEM((BT, BF), jnp.float32),
            pltpu.VMEM((BT, BF), jnp.float32),
            pltpu.VMEM((LPAD, DMODEL), jnp.bfloat16),
        ],
    )
    return pl.pallas_call(
        _ffn_kernel,
        grid_spec=grid_spec,
        out_shape=jax.ShapeDtypeStruct((T, DMODEL), jnp.float32),
        compiler_params=pltpu.CompilerParams(
            dimension_semantics=("arbitrary", "arbitrary", "arbitrary"),
        ),
    )(scalars, xg, W1, W3, W2, tokw3)


@jax.jit
def _moe(x, gate_W, W1, W2, W3):
    gate_w, gate_idx = _router(x, gate_W)

    eid = gate_idx.reshape(-1)
    oh = (eid[:, None] == jnp.arange(E, dtype=jnp.int32)[None, :])
    oh = oh.astype(jnp.int32)
    counts = oh.sum(axis=0)
    rank = jnp.cumsum(oh, axis=0) - oh
    rank_i = (rank * oh).sum(axis=1)
    pc = ((counts + BT - 1) // BT) * BT
    pend = jnp.cumsum(pc)
    pstart = pend - pc
    pos = (pstart[eid] + rank_i).astype(jnp.int32)
    pair_tok = jnp.arange(T * TOPK, dtype=jnp.int32) // TOPK
    wq = jnp.round(gate_w.reshape(-1) * 16384.0).astype(jnp.int32)
    packed = (wq << 16) | pair_tok
    pad_fill = jnp.arange(LPAD, dtype=jnp.int32) % T
    tokw = pad_fill.at[pos].set(packed)
    pos3 = pos.reshape(NW, T // NW, TOPK).transpose(0, 2, 1)
    nvalid = (pend[-1] // BT).astype(jnp.int32)
    te = (jnp.arange(NTILES, dtype=jnp.int32)[:, None] * BT
          >= pend[None, :]).sum(axis=1)
    te = jnp.minimum(te, E - 1).astype(jnp.int32)
    scalars = jnp.concatenate([te, nvalid[None]])
    tokw3 = tokw.reshape(NTILES, 1, BT)

    xg = _dispatch(x, pos3)
    return _ffn(scalars, xg, W1, W3, W2, tokw3)


def kernel(stm, gate_W, W1, W2, W3):
    b, s, h, dh = stm.shape
    x = stm.reshape(b * s, h * dh)
    out = _moe(x, gate_W, W1, W2, W3)
    return out.reshape(b, s, h, dh)


# speedup vs baseline: 1.1904x; 1.1904x over previous
import functools

import jax
import jax.numpy as jnp
from jax import lax
from jax.experimental import pallas as pl
from jax.experimental.pallas import tpu as pltpu
from jax.experimental.pallas import tpu_sc as plsc

B, S, H, D = 1, 2048, 12, 64
DMODEL = H * D
DFF = 2048
E = 8
T = B * S
TOPK = 2

BT = 512
NTILES = (T * TOPK + E * (BT - 1) + BT - 1) // BT
LPAD = NTILES * BT
BF = 1024
NJ = DFF // BF

NC, NS = 2, 16
NW = NC * NS


def _router_kernel(x_ref, gw_ref, w_ref, idx_ref):
    x = x_ref[...]
    logits = jnp.dot(x, gw_ref[...], preferred_element_type=jnp.float32)
    am1 = jnp.argmax(logits, axis=1)[:, None]
    eids = jax.lax.broadcasted_iota(jnp.int32, logits.shape, 1)
    m1 = jnp.max(logits, axis=1, keepdims=True)
    masked = jnp.where(eids == am1, -jnp.inf, logits)
    am2 = jnp.argmax(masked, axis=1)[:, None]
    m2 = jnp.max(masked, axis=1, keepdims=True)
    w1v = 1.0 / (1.0 + jnp.exp(m2 - m1))
    w_ref[...] = jnp.concatenate([w1v, 1.0 - w1v], axis=1)
    idx_ref[...] = jnp.concatenate([am1, am2], axis=1).astype(jnp.int32)


def _router(x, gate_W):
    return pl.pallas_call(
        _router_kernel,
        out_shape=(jax.ShapeDtypeStruct((T, TOPK), jnp.float32),
                   jax.ShapeDtypeStruct((T, TOPK), jnp.int32)),
    )(x, gate_W)


def _dispatch(x, pos3):
    tok_w = T // NW

    mesh = plsc.VectorSubcoreMesh(core_axis_name="c", subcore_axis_name="s")

    @functools.partial(
        pl.kernel, mesh=mesh,
        out_type=jax.ShapeDtypeStruct((LPAD, DMODEL), jnp.float32),
        scratch_types=[
            pltpu.VMEM((TOPK, tok_w), jnp.int32),
            pltpu.VMEM((tok_w, DMODEL), jnp.float32),
            pltpu.SemaphoreType.DMA,
        ],
    )
    def k(x_hbm, pos3_hbm, xg_hbm, idx_v, rows_v, sem):
        wid = lax.axis_index("s") * NC + lax.axis_index("c")
        base = wid * tok_w
        pltpu.sync_copy(pos3_hbm.at[wid], idx_v)
        pltpu.sync_copy(x_hbm.at[pl.ds(base, tok_w)], rows_v)
        c0 = pltpu.async_copy(rows_v, xg_hbm.at[idx_v.at[0]], sem)
        c1 = pltpu.async_copy(rows_v, xg_hbm.at[idx_v.at[1]], sem)
        c0.wait()
        c1.wait()

    return k(x, pos3)


def _ffn_kernel(te_ref, xg_ref, w1_ref, w3_ref, w2_ref, tokw_ref,
                out_ref, w13b_ref, w2b_ref, acc_ref):
    j = pl.program_id(0)
    tl = pl.program_id(1)
    nvalid = te_ref[NTILES]
    prev = te_ref[jnp.maximum(tl - 1, 0)]
    refresh = (tl == 0) | (te_ref[tl] != prev)

    @pl.when((j == 0) & (tl == 0))
    def _():
        out_ref[...] = jnp.zeros_like(out_ref)

    @pl.when(refresh)
    def _():
        w13b_ref[:, :BF] = w1_ref[0].astype(jnp.bfloat16)
        w13b_ref[:, BF:] = w3_ref[0].astype(jnp.bfloat16)
        w2b_ref[...] = w2_ref[0].astype(jnp.bfloat16)

    @pl.when(tl < nvalid)
    def _():
        xr = xg_ref[...]
        xb = jnp.where(jnp.abs(xr) < 1e30, xr, 0.0).astype(jnp.bfloat16)
        gu = jnp.dot(xb, w13b_ref[...], preferred_element_type=jnp.float32)
        g = gu[:, :BF]
        u = gu[:, BF:]
        g = g * jax.nn.sigmoid(g)
        h = (g * u).astype(jnp.bfloat16)
        part = jnp.dot(h, w2b_ref[...], preferred_element_type=jnp.float32)

        sl = pl.ds(tl * BT, BT)
        if NJ > 1:
            @pl.when(j == 0)
            def _():
                acc_ref[sl, :] = part.astype(jnp.bfloat16)

            @pl.when((j > 0) & (j < NJ - 1))
            def _():
                acc_ref[sl, :] += part.astype(jnp.bfloat16)

        @pl.when(j == NJ - 1)
        def _():
            if NJ == 1:
                full = part
            else:
                full = acc_ref[sl, :].astype(jnp.float32) + part
            y = full.astype(jnp.bfloat16)
            v = tokw_ref[0]
            tok = v & 0xFFFF
            wv = (v >> 16).astype(jnp.float32) * (1.0 / 16384.0)
            ti = jax.lax.broadcasted_iota(jnp.int32, (T, BT), 0)
            pt = jnp.where(ti == tok, wv, 0.0).astype(jnp.bfloat16)
            out_ref[...] += jnp.dot(pt, y, preferred_element_type=jnp.float32)


def _ffn(scalars, xg, W1, W3, W2, tokw3):
    grid_spec = pltpu.PrefetchScalarGridSpec(
        num_scalar_prefetch=1,
        grid=(NJ, NTILES),
        in_specs=[
            pl.BlockSpec((BT, DMODEL), lambda j, tl, te: (tl, 0)),
            pl.BlockSpec((1, DMODEL, BF), lambda j, tl, te: (te[tl], 0, j)),
            pl.BlockSpec((1, DMODEL, BF), lambda j, tl, te: (te[tl], 0, j)),
            pl.BlockSpec((1, BF, DMODEL), lambda j, tl, te: (te[tl], j, 0)),
            pl.BlockSpec((1, 1, BT), lambda j, tl, te: (tl, 0, 0)),
        ],
        out_specs=pl.BlockSpec((T, DMODEL), lambda j, tl, te: (0, 0)),
        scratch_shapes=[
            pltpu.VMEM((DMODEL, 2 * BF), jnp.bfloat16),
            pltpu.VMEM((BF, DMODEL), jnp.bfloat16),
            pltpu.VMEM((LPAD, DMODEL), jnp.bfloat16),
        ],
    )
    return pl.pallas_call(
        _ffn_kernel,
        grid_spec=grid_spec,
        out_shape=jax.ShapeDtypeStruct((T, DMODEL), jnp.float32),
        compiler_params=pltpu.CompilerParams(
            dimension_semantics=("arbitrary", "arbitrary"),
        ),
    )(scalars, xg, W1, W3, W2, tokw3)


@jax.jit
def _moe(x, gate_W, W1, W2, W3):
    gate_w, gate_idx = _router(x, gate_W)

    eid = gate_idx.reshape(-1)
    oh = (eid[:, None] == jnp.arange(E, dtype=jnp.int32)[None, :])
    oh = oh.astype(jnp.int32)
    counts = oh.sum(axis=0)
    rank = jnp.cumsum(oh, axis=0) - oh
    rank_i = (rank * oh).sum(axis=1)
    pc = ((counts + BT - 1) // BT) * BT
    pend = jnp.cumsum(pc)
    pstart = pend - pc
    pos = (pstart[eid] + rank_i).astype(jnp.int32)
    pair_tok = jnp.arange(T * TOPK, dtype=jnp.int32) // TOPK
    wq = jnp.round(gate_w.reshape(-1) * 16384.0).astype(jnp.int32)
    packed = (wq << 16) | pair_tok
    pad_fill = jnp.arange(LPAD, dtype=jnp.int32) % T
    tokw = pad_fill.at[pos].set(packed)
    pos3 = pos.reshape(NW, T // NW, TOPK).transpose(0, 2, 1)
    nvalid = (pend[-1] // BT).astype(jnp.int32)
    te = (jnp.arange(NTILES, dtype=jnp.int32)[:, None] * BT
          >= pend[None, :]).sum(axis=1)
    te = jnp.minimum(te, E - 1).astype(jnp.int32)
    scalars = jnp.concatenate([te, nvalid[None]])
    tokw3 = tokw.reshape(NTILES, 1, BT)

    xg = _dispatch(x, pos3)
    return _ffn(scalars, xg, W1, W3, W2, tokw3)


def kernel(stm, gate_W, W1, W2, W3):
    b, s, h, dh = stm.shape
    x = stm.reshape(b * s, h * dh)
    out = _moe(x, gate_W, W1, W2, W3)
    return out.reshape(b, s, h, dh)


# speedup vs baseline: 1.2637x; 1.0616x over previous
import functools

import jax
import jax.numpy as jnp
from jax import lax
from jax.experimental import pallas as pl
from jax.experimental.pallas import tpu as pltpu
from jax.experimental.pallas import tpu_sc as plsc

B, S, H, D = 1, 2048, 12, 64
DMODEL = H * D
DFF = 2048
E = 8
T = B * S
TOPK = 2

BT = 512
NTILES = (T * TOPK + E * (BT - 1) + BT - 1) // BT
LPAD = NTILES * BT
BF = 1024
NJ = DFF // BF

NC, NS = 2, 16
NW = NC * NS


def _router_kernel(x_ref, gw_ref, pos_ref, pk_ref, cnt_ref):
    x = x_ref[...]
    logits = jnp.dot(x, gw_ref[...], preferred_element_type=jnp.float32)
    am1 = jnp.argmax(logits, axis=1)[:, None]
    eids = jax.lax.broadcasted_iota(jnp.int32, logits.shape, 1)
    m1 = jnp.max(logits, axis=1, keepdims=True)
    masked = jnp.where(eids == am1, -jnp.inf, logits)
    am2 = jnp.argmax(masked, axis=1)[:, None]
    m2 = jnp.max(masked, axis=1, keepdims=True)
    w1v = 1.0 / (1.0 + jnp.exp(m2 - m1))

    oh1 = (eids == am1).astype(jnp.int32)
    oh2 = (eids == am2).astype(jnp.int32)
    s = oh1 + oh2
    c = s
    d = 1
    while d < T:
        shifted = jnp.concatenate(
            [jnp.zeros((d, E), jnp.int32), c[:T - d, :]], axis=0)
        c = c + shifted
        d *= 2
    cnt_ref[...] = c[T - 1:T, :]
    cex = c - s
    pc = ((c[T - 1:T, :] + BT - 1) // BT) * BT
    ps = pc
    d = 1
    while d < E:
        ps = ps + jnp.concatenate(
            [jnp.zeros((1, d), jnp.int32), ps[:, :E - d]], axis=1)
        d *= 2
    pstart = ps - pc
    base0 = cex + pstart
    pos0 = jnp.sum(oh1 * base0, axis=1, keepdims=True)
    pos1 = jnp.sum(oh2 * (base0 + oh1), axis=1, keepdims=True)
    pos_ref[...] = jnp.concatenate([pos0, pos1], axis=1)
    tok = jax.lax.broadcasted_iota(jnp.int32, (T, 1), 0)
    wq1 = jnp.round(w1v * 16384.0).astype(jnp.int32)
    wq2 = 16384 - wq1
    pk_ref[...] = jnp.concatenate(
        [(wq1 << 16) | tok, (wq2 << 16) | tok], axis=1)


def _router(x, gate_W):
    return pl.pallas_call(
        _router_kernel,
        out_shape=(jax.ShapeDtypeStruct((T, TOPK), jnp.int32),
                   jax.ShapeDtypeStruct((T, TOPK), jnp.int32),
                   jax.ShapeDtypeStruct((1, E), jnp.int32)),
    )(x, gate_W)


def _dispatch(x, pos3):
    tok_w = T // NW

    mesh = plsc.VectorSubcoreMesh(core_axis_name="c", subcore_axis_name="s")

    @functools.partial(
        pl.kernel, mesh=mesh,
        out_type=jax.ShapeDtypeStruct((LPAD, DMODEL), jnp.float32),
        scratch_types=[
            pltpu.VMEM((TOPK, tok_w), jnp.int32),
            pltpu.VMEM((tok_w, DMODEL), jnp.float32),
            pltpu.SemaphoreType.DMA,
        ],
    )
    def k(x_hbm, pos3_hbm, xg_hbm, idx_v, rows_v, sem):
        wid = lax.axis_index("s") * NC + lax.axis_index("c")
        base = wid * tok_w
        pltpu.sync_copy(pos3_hbm.at[wid], idx_v)
        pltpu.sync_copy(x_hbm.at[pl.ds(base, tok_w)], rows_v)
        c0 = pltpu.async_copy(rows_v, xg_hbm.at[idx_v.at[0]], sem)
        c1 = pltpu.async_copy(rows_v, xg_hbm.at[idx_v.at[1]], sem)
        c0.wait()
        c1.wait()

    return k(x, pos3)


def _ffn_kernel(te_ref, xg_ref, w1_ref, w3_ref, w2_ref, tokw_ref,
                out_ref, w13b_ref, w2b_ref, acc_ref):
    j = pl.program_id(0)
    tl = pl.program_id(1)
    nvalid = te_ref[NTILES]
    prev = te_ref[jnp.maximum(tl - 1, 0)]
    refresh = (tl == 0) | (te_ref[tl] != prev)

    @pl.when((j == 0) & (tl == 0))
    def _():
        out_ref[...] = jnp.zeros_like(out_ref)

    @pl.when(refresh)
    def _():
        w13b_ref[:, :BF] = w1_ref[0].astype(jnp.bfloat16)
        w13b_ref[:, BF:] = w3_ref[0].astype(jnp.bfloat16)
        w2b_ref[...] = w2_ref[0].astype(jnp.bfloat16)

    @pl.when(tl < nvalid)
    def _():
        xr = xg_ref[...]
        xb = jnp.where(jnp.abs(xr) < 1e30, xr, 0.0).astype(jnp.bfloat16)
        gu = jnp.dot(xb, w13b_ref[...], preferred_element_type=jnp.float32)
        g = gu[:, :BF]
        u = gu[:, BF:]
        g = g * jax.nn.sigmoid(g)
        h = (g * u).astype(jnp.bfloat16)
        part = jnp.dot(h, w2b_ref[...], preferred_element_type=jnp.float32)

        sl = pl.ds(tl * BT, BT)
        if NJ > 1:
            @pl.when(j == 0)
            def _():
                acc_ref[sl, :] = part.astype(jnp.bfloat16)

            @pl.when((j > 0) & (j < NJ - 1))
            def _():
                acc_ref[sl, :] += part.astype(jnp.bfloat16)

        @pl.when(j == NJ - 1)
        def _():
            if NJ == 1:
                full = part
            else:
                full = acc_ref[sl, :].astype(jnp.float32) + part
            y = full.astype(jnp.bfloat16)
            v = tokw_ref[0]
            tok = v & 0xFFFF
            wv = (v >> 16).astype(jnp.float32) * (1.0 / 16384.0)
            ti = jax.lax.broadcasted_iota(jnp.int32, (T, BT), 0)
            pt = jnp.where(ti == tok, wv, 0.0).astype(jnp.bfloat16)
            out_ref[...] += jnp.dot(pt, y, preferred_element_type=jnp.float32)


def _ffn(scalars, xg, W1, W3, W2, tokw3):
    grid_spec = pltpu.PrefetchScalarGridSpec(
        num_scalar_prefetch=1,
        grid=(NJ, NTILES),
        in_specs=[
            pl.BlockSpec((BT, DMODEL), lambda j, tl, te: (tl, 0)),
            pl.BlockSpec((1, DMODEL, BF), lambda j, tl, te: (te[tl], 0, j)),
            pl.BlockSpec((1, DMODEL, BF), lambda j, tl, te: (te[tl], 0, j)),
            pl.BlockSpec((1, BF, DMODEL), lambda j, tl, te: (te[tl], j, 0)),
            pl.BlockSpec((1, 1, BT), lambda j, tl, te: (tl, 0, 0)),
        ],
        out_specs=pl.BlockSpec((T, DMODEL), lambda j, tl, te: (0, 0)),
        scratch_shapes=[
            pltpu.VMEM((DMODEL, 2 * BF), jnp.bfloat16),
            pltpu.VMEM((BF, DMODEL), jnp.bfloat16),
            pltpu.VMEM((LPAD, DMODEL), jnp.bfloat16),
        ],
    )
    return pl.pallas_call(
        _ffn_kernel,
        grid_spec=grid_spec,
        out_shape=jax.ShapeDtypeStruct((T, DMODEL), jnp.float32),
        compiler_params=pltpu.CompilerParams(
            dimension_semantics=("arbitrary", "arbitrary"),
        ),
    )(scalars, xg, W1, W3, W2, tokw3)


@jax.jit
def _moe(x, gate_W, W1, W2, W3):
    pos, packed, counts = _router(x, gate_W)

    pc = ((counts[0] + BT - 1) // BT) * BT
    pend = jnp.cumsum(pc)
    pos_flat = pos.reshape(-1)
    pad_fill = jnp.arange(LPAD, dtype=jnp.int32) % T
    tokw = pad_fill.at[pos_flat].set(packed.reshape(-1))
    pos3 = pos.reshape(NW, T // NW, TOPK).transpose(0, 2, 1)
    nvalid = (pend[-1] // BT).astype(jnp.int32)
    te = (jnp.arange(NTILES, dtype=jnp.int32)[:, None] * BT
          >= pend[None, :]).sum(axis=1)
    te = jnp.minimum(te, E - 1).astype(jnp.int32)
    scalars = jnp.concatenate([te, nvalid[None]])
    tokw3 = tokw.reshape(NTILES, 1, BT)

    xg = _dispatch(x, pos3)
    return _ffn(scalars, xg, W1, W3, W2, tokw3)


def kernel(stm, gate_W, W1, W2, W3):
    b, s, h, dh = stm.shape
    x = stm.reshape(b * s, h * dh)
    out = _moe(x, gate_W, W1, W2, W3)
    return out.reshape(b, s, h, dh)
